# Initial kernel scaffold; baseline (speedup 1.0000x reference)
import jax
import jax.numpy as jnp
from jax.experimental import pallas as pl


def kernel(streams, W_qkv, W_o, norm1_w, norm2_w, hc1_f1, hc1_f2, hc1_pre, hc1_post, hc2_f1, hc2_f2, hc2_pre, hc2_post, Wg, Wu, Wd, sAg, sBg, sAu, sBu, sAd, sBd, eAg, eBg, eAu, eBu, eAd, eBd, Wr, expert_bias):
    raise NotImplementedError("write your pallas kernel here")


# speedup vs baseline: 1.1328x; 1.1328x over previous
import functools

import jax
import jax.numpy as jnp
import numpy as np
from jax.experimental import pallas as pl

D = 768; H = 12; DH = 64; E = 8; K = 2; R = 8; DF = 2048; NS = 4; B = 1; L = 2048; MAXLEN = 4096
LB = 256
NLB = L // LB
SCALE = 1.0 / R
NEG = -1e30


def _rms(x, w):
    return x * jax.lax.rsqrt(jnp.mean(x * x, axis=-1, keepdims=True) + 1e-6) * w


def _rms_matmul_kernel(x_ref, w_ref, wm_ref, o_ref):
    xn = _rms(x_ref[...], w_ref[...])
    o_ref[...] = jnp.dot(xn, wm_ref[...], preferred_element_type=jnp.float32)


def _matmul_kernel(x_ref, wm_ref, o_ref):
    o_ref[...] = jnp.dot(x_ref[...], wm_ref[...], preferred_element_type=jnp.float32)


def _attn_kernel(q_ref, k_ref, v_ref, o_ref):
    qb = pl.program_id(1)
    q = q_ref[0]
    k = k_ref[0]
    v = v_ref[0]
    s = jax.lax.dot_general(q, k, (((1,), (1,)), ((), ())),
                            preferred_element_type=jnp.float32)
    s = s * (1.0 / np.sqrt(DH))
    row = jax.lax.broadcasted_iota(jnp.int32, (LB, L), 0) + qb * LB
    col = jax.lax.broadcasted_iota(jnp.int32, (LB, L), 1)
    s = jnp.where(col <= row, s, NEG)
    m = jnp.max(s, axis=1, keepdims=True)
    p = jnp.exp(s - m)
    p = p / jnp.sum(p, axis=1, keepdims=True)
    o_ref[0] = jnp.dot(p, v, preferred_element_type=jnp.float32)


def _ffn_kernel(x_ref, nw_ref, wr_ref, eb_ref, mexp_ref,
                wg_ref, wu_ref, wd_ref,
                sag_ref, sbg_ref, sau_ref, sbu_ref, sad_ref, sbd_ref,
                eag_ref, ebg_ref, eau_ref, ebu_ref, ead_ref, ebd_ref,
                o_ref):
    xn = _rms(x_ref[...], nw_ref[...])

    logits = jnp.dot(xn, wr_ref[...], preferred_element_type=jnp.float32)
    scores = jax.nn.sigmoid(logits)
    biased = scores + eb_ref[...]
    iota = jax.lax.broadcasted_iota(jnp.int32, (LB, E), 1)
    m1 = jnp.max(biased, axis=1, keepdims=True)
    i1 = jnp.min(jnp.where(biased == m1, iota, E), axis=1, keepdims=True)
    oh1 = iota == i1
    tk1 = jnp.sum(jnp.where(oh1, scores, 0.0), axis=1, keepdims=True)
    b2 = jnp.where(oh1, NEG, biased)
    m2 = jnp.max(b2, axis=1, keepdims=True)
    i2 = jnp.min(jnp.where(b2 == m2, iota, E), axis=1, keepdims=True)
    oh2 = iota == i2
    tk2 = jnp.sum(jnp.where(oh2, scores, 0.0), axis=1, keepdims=True)
    den = tk1 + tk2 + 1e-8
    ew = jnp.where(oh1, tk1 / den, 0.0) + jnp.where(oh2, tk2 / den, 0.0)
    wrep = jnp.dot(ew, mexp_ref[...], preferred_element_type=jnp.float32)

    def lora3(z, sa, sb, ea, eb):
        shared = jnp.dot(jnp.dot(z, sa, preferred_element_type=jnp.float32), sb,
                         preferred_element_type=jnp.float32)
        za = jnp.dot(z, ea, preferred_element_type=jnp.float32) * wrep
        expert = jnp.dot(za, eb, preferred_element_type=jnp.float32)
        return (shared + expert) * SCALE

    gate = jnp.dot(xn, wg_ref[...], preferred_element_type=jnp.float32) \
        + lora3(xn, sag_ref[...], sbg_ref[...], eag_ref[...], ebg_ref[...])
    up = jnp.dot(xn, wu_ref[...], preferred_element_type=jnp.float32) \
        + lora3(xn, sau_ref[...], sbu_ref[...], eau_ref[...], ebu_ref[...])
    h = (gate * jax.nn.sigmoid(gate)) * up
    down = jnp.dot(h, wd_ref[...], preferred_element_type=jnp.float32) \
        + lora3(h, sad_ref[...], sbd_ref[...], ead_ref[...], ebd_ref[...])
    o_ref[...] = down


def _rms_matmul(x, w, wm):
    n = wm.shape[1]
    return pl.pallas_call(
        _rms_matmul_kernel,
        grid=(NLB,),
        in_specs=[
            pl.BlockSpec((LB, D), lambda i: (i, 0)),
            pl.BlockSpec((1, D), lambda i: (0, 0)),
            pl.BlockSpec((D, n), lambda i: (0, 0)),
        ],
        out_specs=pl.BlockSpec((LB, n), lambda i: (i, 0)),
        out_shape=jax.ShapeDtypeStruct((L, n), jnp.float32),
    )(x, w, wm)


def _matmul(x, wm):
    m, k = x.shape
    n = wm.shape[1]
    return pl.pallas_call(
        _matmul_kernel,
        grid=(m // LB,),
        in_specs=[
            pl.BlockSpec((LB, k), lambda i: (i, 0)),
            pl.BlockSpec((k, n), lambda i: (0, 0)),
        ],
        out_specs=pl.BlockSpec((LB, n), lambda i: (i, 0)),
        out_shape=jax.ShapeDtypeStruct((m, n), jnp.float32),
    )(x, wm)


def _attention(q, k, v):
    return pl.pallas_call(
        _attn_kernel,
        grid=(H, NLB),
        in_specs=[
            pl.BlockSpec((1, LB, DH), lambda h, i: (h, i, 0)),
            pl.BlockSpec((1, L, DH), lambda h, i: (h, 0, 0)),
            pl.BlockSpec((1, L, DH), lambda h, i: (h, 0, 0)),
        ],
        out_specs=pl.BlockSpec((1, LB, DH), lambda h, i: (h, i, 0)),
        out_shape=jax.ShapeDtypeStruct((H, L, DH), jnp.float32),
    )(q, k, v)


def _ffn(x, nw, wr, eb, mexp, wg, wu, wd, sag, sbg, sau, sbu, sad, sbd,
         eagf, ebgf, eauf, ebuf, eadf, ebdf):
    full = lambda a: pl.BlockSpec(a.shape, functools.partial(lambda nd, i: (0,) * nd, a.ndim))
    return pl.pallas_call(
        _ffn_kernel,
        grid=(NLB,),
        in_specs=[
            pl.BlockSpec((LB, D), lambda i: (i, 0)),
            full(nw), full(wr), full(eb), full(mexp),
            full(wg), full(wu), full(wd),
            full(sag), full(sbg), full(sau), full(sbu), full(sad), full(sbd),
            full(eagf), full(ebgf), full(eauf), full(ebuf), full(eadf), full(ebdf),
        ],
        out_specs=pl.BlockSpec((LB, D), lambda i: (i, 0)),
        out_shape=jax.ShapeDtypeStruct((L, D), jnp.float32),
    )(x, nw, wr, eb, mexp, wg, wu, wd, sag, sbg, sau, sbu, sad, sbd,
      eagf, ebgf, eauf, ebuf, eadf, ebdf)


def _rope_tables():
    pos = jnp.arange(MAXLEN, dtype=jnp.float32)
    freqs = 1.0 / (10000.0 ** (jnp.arange(0, DH, 2, dtype=jnp.float32) / DH))
    ang = pos[:, None] * freqs[None, :]
    return jnp.cos(ang), jnp.sin(ang)


def _apply_rope(x, cos, sin):
    dh = DH // 2
    c = cos[:L, :dh][None]
    s = sin[:L, :dh][None]
    x1, x2 = x[..., :dh], x[..., dh:]
    return jnp.concatenate([x1 * c - x2 * s, x2 * c + x1 * s], axis=-1)


def _hres(f1, f2):
    I2 = jnp.eye(2, dtype=jnp.float32)
    S2 = jnp.array([[0.0, 1.0], [1.0, 0.0]], dtype=jnp.float32)
    a1 = jax.nn.softmax(f1)[0]
    a2 = jax.nn.softmax(f2)[0]
    U1 = a1 * I2 + (1 - a1) * S2
    U2 = a2 * I2 + (1 - a2) * S2
    return jnp.kron(U1, U2)


def kernel(streams, W_qkv, W_o, norm1_w, norm2_w, hc1_f1, hc1_f2, hc1_pre,
           hc1_post, hc2_f1, hc2_f2, hc2_pre, hc2_post, Wg, Wu, Wd, sAg, sBg,
           sAu, sBu, sAd, sBd, eAg, eBg, eAu, eBu, eAd, eBd, Wr, expert_bias):
    cos, sin = _rope_tables()

    Hres1 = _hres(hc1_f1, hc1_f2)
    mixed1 = jnp.einsum('ij,bljd->blid', Hres1, streams)
    pw1 = jax.nn.softmax(hc1_pre)
    bi1 = jnp.einsum('n,blnd->bld', pw1, streams)[0]

    qkv = _rms_matmul(bi1, norm1_w[None, :], W_qkv)
    qkv = qkv.reshape(L, 3, H, DH)
    q = jnp.transpose(qkv[:, 0], (1, 0, 2))
    k = jnp.transpose(qkv[:, 1], (1, 0, 2))
    v = jnp.transpose(qkv[:, 2], (1, 0, 2))
    q = _apply_rope(q, cos, sin)
    k = _apply_rope(k, cos, sin)
    o = _attention(q, k, v)
    o = jnp.transpose(o, (1, 0, 2)).reshape(L, D)
    bo1 = _matmul(o, W_o)

    po1 = jax.nn.softmax(hc1_post)
    s1 = mixed1 + bo1[None, :, None, :] * po1[None, None, :, None]

    Hres2 = _hres(hc2_f1, hc2_f2)
    mixed2 = jnp.einsum('ij,bljd->blid', Hres2, s1)
    pw2 = jax.nn.softmax(hc2_pre)
    bi2 = jnp.einsum('n,blnd->bld', pw2, s1)[0]

    mexp = jnp.repeat(jnp.eye(E, dtype=jnp.float32), R, axis=1)
    eAgf = jnp.transpose(eAg, (1, 0, 2)).reshape(D, E * R)
    eBgf = eBg.reshape(E * R, DF)
    eAuf = jnp.transpose(eAu, (1, 0, 2)).reshape(D, E * R)
    eBuf = eBu.reshape(E * R, DF)
    eAdf = jnp.transpose(eAd, (1, 0, 2)).reshape(DF, E * R)
    eBdf = eBd.reshape(E * R, D)

    bo2 = _ffn(bi2, norm2_w[None, :], Wr, expert_bias[None, :], mexp,
               Wg, Wu, Wd, sAg, sBg, sAu, sBu, sAd, sBd,
               eAgf, eBgf, eAuf, eBuf, eAdf, eBdf)

    po2 = jax.nn.softmax(hc2_post)
    s2 = mixed2 + bo2[None, :, None, :] * po2[None, None, :, None]
    return s2


# speedup vs baseline: 1.2448x; 1.0989x over previous
import functools

import jax
import jax.numpy as jnp
import numpy as np
from jax.experimental import pallas as pl

D = 768; H = 12; DH = 64; E = 8; K = 2; R = 8; DF = 2048; NS = 4; B = 1; L = 2048; MAXLEN = 4096
LB = 256
NLB = L // LB
SCALE = 1.0 / R
NEG = -1e30


def _rms(x, w):
    return x * jax.lax.rsqrt(jnp.mean(x * x, axis=-1, keepdims=True) + 1e-6) * w


def _qkv_kernel(st_ref, c1_ref, w_ref, wm_ref, o_ref):
    c = c1_ref[...]
    bi = (st_ref[:, 0, :] * c[0, 0] + st_ref[:, 1, :] * c[0, 1]
          + st_ref[:, 2, :] * c[0, 2] + st_ref[:, 3, :] * c[0, 3])
    xn = _rms(bi, w_ref[...])
    o_ref[...] = jnp.dot(xn, wm_ref[...], preferred_element_type=jnp.float32)


def _matmul_kernel(x_ref, wm_ref, o_ref):
    o_ref[...] = jnp.dot(x_ref[...], wm_ref[...], preferred_element_type=jnp.float32)


def _attn_kernel(q_ref, k_ref, v_ref, o_ref):
    qb = pl.program_id(1)
    q = q_ref[0]
    k = k_ref[0]
    v = v_ref[0]
    s = jax.lax.dot_general(q, k, (((1,), (1,)), ((), ())),
                            preferred_element_type=jnp.float32)
    s = s * (1.0 / np.sqrt(DH))
    row = jax.lax.broadcasted_iota(jnp.int32, (LB, L), 0) + qb * LB
    col = jax.lax.broadcasted_iota(jnp.int32, (LB, L), 1)
    s = jnp.where(col <= row, s, NEG)
    m = jnp.max(s, axis=1, keepdims=True)
    p = jnp.exp(s - m)
    r = 1.0 / jnp.sum(p, axis=1, keepdims=True)
    o_ref[0] = jnp.dot(p, v, preferred_element_type=jnp.float32) * r


def _ffn_kernel(st_ref, bo1_ref, c2_ref, al_ref, g_ref, g1_ref, po2_ref,
                nw_ref, wr_ref, eb_ref, mexp_ref,
                wg_ref, wu_ref, wd_ref,
                sag_ref, sbg_ref, sau_ref, sbu_ref, sad_ref, sbd_ref,
                eag_ref, ebg_ref, eau_ref, ebu_ref, ead_ref, ebd_ref,
                o_ref):
    c2 = c2_ref[...]
    bo1 = bo1_ref[...]
    x = (st_ref[:, 0, :] * c2[0, 0] + st_ref[:, 1, :] * c2[0, 1]
         + st_ref[:, 2, :] * c2[0, 2] + st_ref[:, 3, :] * c2[0, 3]
         + bo1 * al_ref[0, 0])
    xn = _rms(x, nw_ref[...])

    logits = jnp.dot(xn, wr_ref[...], preferred_element_type=jnp.float32)
    scores = jax.nn.sigmoid(logits)
    biased = scores + eb_ref[...]
    iota = jax.lax.broadcasted_iota(jnp.int32, (LB, E), 1)
    m1 = jnp.max(biased, axis=1, keepdims=True)
    i1 = jnp.min(jnp.where(biased == m1, iota, E), axis=1, keepdims=True)
    oh1 = iota == i1
    tk1 = jnp.sum(jnp.where(oh1, scores, 0.0), axis=1, keepdims=True)
    b2 = jnp.where(oh1, NEG, biased)
    m2 = jnp.max(b2, axis=1, keepdims=True)
    i2 = jnp.min(jnp.where(b2 == m2, iota, E), axis=1, keepdims=True)
    oh2 = iota == i2
    tk2 = jnp.sum(jnp.where(oh2, scores, 0.0), axis=1, keepdims=True)
    den = tk1 + tk2 + 1e-8
    ew = jnp.where(oh1, tk1 / den, 0.0) + jnp.where(oh2, tk2 / den, 0.0)
    wrep = jnp.dot(ew, mexp_ref[...], preferred_element_type=jnp.float32)

    def lora3(z, sa, sb, ea, eb):
        shared = jnp.dot(jnp.dot(z, sa, preferred_element_type=jnp.float32), sb,
                         preferred_element_type=jnp.float32)
        za = jnp.dot(z, ea, preferred_element_type=jnp.float32) * wrep
        expert = jnp.dot(za, eb, preferred_element_type=jnp.float32)
        return (shared + expert) * SCALE

    gate = jnp.dot(xn, wg_ref[...], preferred_element_type=jnp.float32) \
        + lora3(xn, sag_ref[...], sbg_ref[...], eag_ref[...], ebg_ref[...])
    up = jnp.dot(xn, wu_ref[...], preferred_element_type=jnp.float32) \
        + lora3(xn, sau_ref[...], sbu_ref[...], eau_ref[...], ebu_ref[...])
    h = (gate * jax.nn.sigmoid(gate)) * up
    down = jnp.dot(h, wd_ref[...], preferred_element_type=jnp.float32) \
        + lora3(h, sad_ref[...], sbd_ref[...], ead_ref[...], ebd_ref[...])

    g = g_ref[...]
    g1 = g1_ref[...]
    po2 = po2_ref[...]
    for n in range(NS):
        o_ref[:, n, :] = (st_ref[:, 0, :] * g[n, 0] + st_ref[:, 1, :] * g[n, 1]
                          + st_ref[:, 2, :] * g[n, 2] + st_ref[:, 3, :] * g[n, 3]
                          + bo1 * g1[0, n] + down * po2[0, n])


def _qkv(st, c1, w, wm):
    n = wm.shape[1]
    return pl.pallas_call(
        _qkv_kernel,
        grid=(NLB,),
        in_specs=[
            pl.BlockSpec((LB, NS, D), lambda i: (i, 0, 0)),
            pl.BlockSpec((1, NS), lambda i: (0, 0)),
            pl.BlockSpec((1, D), lambda i: (0, 0)),
            pl.BlockSpec((D, n), lambda i: (0, 0)),
        ],
        out_specs=pl.BlockSpec((LB, n), lambda i: (i, 0)),
        out_shape=jax.ShapeDtypeStruct((L, n), jnp.float32),
    )(st, c1, w, wm)


def _matmul(x, wm):
    m, k = x.shape
    n = wm.shape[1]
    return pl.pallas_call(
        _matmul_kernel,
        grid=(m // LB,),
        in_specs=[
            pl.BlockSpec((LB, k), lambda i: (i, 0)),
            pl.BlockSpec((k, n), lambda i: (0, 0)),
        ],
        out_specs=pl.BlockSpec((LB, n), lambda i: (i, 0)),
        out_shape=jax.ShapeDtypeStruct((m, n), jnp.float32),
    )(x, wm)


def _attention(q, k, v):
    return pl.pallas_call(
        _attn_kernel,
        grid=(H, NLB),
        in_specs=[
            pl.BlockSpec((1, LB, DH), lambda h, i: (h, i, 0)),
            pl.BlockSpec((1, L, DH), lambda h, i: (h, 0, 0)),
            pl.BlockSpec((1, L, DH), lambda h, i: (h, 0, 0)),
        ],
        out_specs=pl.BlockSpec((1, LB, DH), lambda h, i: (h, i, 0)),
        out_shape=jax.ShapeDtypeStruct((H, L, DH), jnp.float32),
    )(q, k, v)


def _ffn(st, bo1, c2, al, g, g1, po2, nw, wr, eb, mexp,
         wg, wu, wd, sag, sbg, sau, sbu, sad, sbd,
         eagf, ebgf, eauf, ebuf, eadf, ebdf):
    full = lambda a: pl.BlockSpec(a.shape, functools.partial(lambda nd, i: (0,) * nd, a.ndim))
    return pl.pallas_call(
        _ffn_kernel,
        grid=(NLB,),
        in_specs=[
            pl.BlockSpec((LB, NS, D), lambda i: (i, 0, 0)),
            pl.BlockSpec((LB, D), lambda i: (i, 0)),
            full(c2), full(al), full(g), full(g1), full(po2),
            full(nw), full(wr), full(eb), full(mexp),
            full(wg), full(wu), full(wd),
            full(sag), full(sbg), full(sau), full(sbu), full(sad), full(sbd),
            full(eagf), full(ebgf), full(eauf), full(ebuf), full(eadf), full(ebdf),
        ],
        out_specs=pl.BlockSpec((LB, NS, D), lambda i: (i, 0, 0)),
        out_shape=jax.ShapeDtypeStruct((L, NS, D), jnp.float32),
    )(st, bo1, c2, al, g, g1, po2, nw, wr, eb, mexp,
      wg, wu, wd, sag, sbg, sau, sbu, sad, sbd,
      eagf, ebgf, eauf, ebuf, eadf, ebdf)


def _rope_tables():
    pos = jnp.arange(MAXLEN, dtype=jnp.float32)
    freqs = 1.0 / (10000.0 ** (jnp.arange(0, DH, 2, dtype=jnp.float32) / DH))
    ang = pos[:, None] * freqs[None, :]
    return jnp.cos(ang), jnp.sin(ang)


def _apply_rope(x, cos, sin):
    dh = DH // 2
    c = cos[:L, :dh][None]
    s = sin[:L, :dh][None]
    x1, x2 = x[..., :dh], x[..., dh:]
    return jnp.concatenate([x1 * c - x2 * s, x2 * c + x1 * s], axis=-1)


def _hres(f1, f2):
    I2 = jnp.eye(2, dtype=jnp.float32)
    S2 = jnp.array([[0.0, 1.0], [1.0, 0.0]], dtype=jnp.float32)
    a1 = jax.nn.softmax(f1)[0]
    a2 = jax.nn.softmax(f2)[0]
    U1 = a1 * I2 + (1 - a1) * S2
    U2 = a2 * I2 + (1 - a2) * S2
    return jnp.kron(U1, U2)


def kernel(streams, W_qkv, W_o, norm1_w, norm2_w, hc1_f1, hc1_f2, hc1_pre,
           hc1_post, hc2_f1, hc2_f2, hc2_pre, hc2_post, Wg, Wu, Wd, sAg, sBg,
           sAu, sBu, sAd, sBd, eAg, eBg, eAu, eBu, eAd, eBd, Wr, expert_bias):
    cos, sin = _rope_tables()
    st = streams[0]

    Hres1 = _hres(hc1_f1, hc1_f2)
    Hres2 = _hres(hc2_f1, hc2_f2)
    pw1 = jax.nn.softmax(hc1_pre)
    pw2 = jax.nn.softmax(hc2_pre)
    po1 = jax.nn.softmax(hc1_post)
    po2 = jax.nn.softmax(hc2_post)
    c1 = pw1[None, :]
    c2 = (Hres1.T @ pw2)[None, :]
    al = (pw2 @ po1)[None, None]
    G = Hres2 @ Hres1
    g1 = (Hres2 @ po1)[None, :]

    qkv = _qkv(st, c1, norm1_w[None, :], W_qkv)
    qkv = qkv.reshape(L, 3, H, DH)
    q = jnp.transpose(qkv[:, 0], (1, 0, 2))
    k = jnp.transpose(qkv[:, 1], (1, 0, 2))
    v = jnp.transpose(qkv[:, 2], (1, 0, 2))
    q = _apply_rope(q, cos, sin)
    k = _apply_rope(k, cos, sin)
    o = _attention(q, k, v)
    o = jnp.transpose(o, (1, 0, 2)).reshape(L, D)
    bo1 = _matmul(o, W_o)

    mexp = jnp.repeat(jnp.eye(E, dtype=jnp.float32), R, axis=1)
    eAgf = jnp.transpose(eAg, (1, 0, 2)).reshape(D, E * R)
    eBgf = eBg.reshape(E * R, DF)
    eAuf = jnp.transpose(eAu, (1, 0, 2)).reshape(D, E * R)
    eBuf = eBu.reshape(E * R, DF)
    eAdf = jnp.transpose(eAd, (1, 0, 2)).reshape(DF, E * R)
    eBdf = eBd.reshape(E * R, D)

    s2 = _ffn(st, bo1, c2, al, G, g1, po2[None, :],
              norm2_w[None, :], Wr, expert_bias[None, :], mexp,
              Wg, Wu, Wd, sAg, sBg, sAu, sBu, sAd, sBd,
              eAgf, eBgf, eAuf, eBuf, eAdf, eBdf)
    return s2[None]


# speedup vs baseline: 2.0024x; 1.6086x over previous
import functools

import jax
import jax.numpy as jnp
import numpy as np
from jax.experimental import pallas as pl

D = 768; H = 12; DH = 64; E = 8; K = 2; R = 8; DF = 2048; NS = 4; B = 1; L = 2048; MAXLEN = 4096
LB = 256
NLB = L // LB
SCALE = 1.0 / R
NEG = -1e30


def _rms(x, w):
    return x * jax.lax.rsqrt(jnp.mean(x * x, axis=-1, keepdims=True) + 1e-6) * w


def _qkv_kernel(st_ref, c1_ref, w_ref, wq_ref, wqp_ref, wk_ref, wkp_ref,
                wv_ref, cos_ref, sin_ref, q_ref, k_ref, v_ref):
    c = c1_ref[...]
    bi = (st_ref[:, 0, :] * c[0, 0] + st_ref[:, 1, :] * c[0, 1]
          + st_ref[:, 2, :] * c[0, 2] + st_ref[:, 3, :] * c[0, 3])
    xn = _rms(bi, w_ref[...])
    cosb = cos_ref[...]
    sinb = sin_ref[...]
    qa = jnp.dot(xn, wq_ref[...], preferred_element_type=jnp.float32)
    qb = jnp.dot(xn, wqp_ref[...], preferred_element_type=jnp.float32)
    q_ref[...] = (qa * cosb + qb * sinb) * (1.0 / np.sqrt(DH))
    ka = jnp.dot(xn, wk_ref[...], preferred_element_type=jnp.float32)
    kb = jnp.dot(xn, wkp_ref[...], preferred_element_type=jnp.float32)
    k_ref[...] = ka * cosb + kb * sinb
    v_ref[...] = jnp.dot(xn, wv_ref[...], preferred_element_type=jnp.float32)


def _attn_kernel(q_ref, k_ref, v_ref, wo_ref, o_ref):
    i = pl.program_id(0)
    row = jax.lax.broadcasted_iota(jnp.int32, (LB, L), 0) + i * LB
    col = jax.lax.broadcasted_iota(jnp.int32, (LB, L), 1)
    neg = jnp.where(col <= row, 0.0, NEG)
    outs = []
    for h in range(H):
        sl = slice(h * DH, (h + 1) * DH)
        q = q_ref[:, sl]
        k = k_ref[:, sl]
        s = jax.lax.dot_general(q, k, (((1,), (1,)), ((), ())),
                                preferred_element_type=jnp.float32) + neg
        m = jnp.max(s, axis=1, keepdims=True)
        p = jnp.exp(s - m)
        r = 1.0 / jnp.sum(p, axis=1, keepdims=True)
        outs.append(jnp.dot(p, v_ref[:, sl], preferred_element_type=jnp.float32) * r)
    o_all = jnp.concatenate(outs, axis=1)
    o_ref[...] = jnp.dot(o_all, wo_ref[...], preferred_element_type=jnp.float32)


def _ffn_kernel(st_ref, bo1_ref, c2_ref, al_ref, g_ref, g1_ref, po2_ref,
                nw_ref, wr_ref, eb_ref, mexp_ref,
                wg_ref, wu_ref, wd_ref,
                sag_ref, sbg_ref, sau_ref, sbu_ref, sad_ref, sbd_ref,
                eag_ref, ebg_ref, eau_ref, ebu_ref, ead_ref, ebd_ref,
                o_ref):
    c2 = c2_ref[...]
    bo1 = bo1_ref[...]
    x = (st_ref[:, 0, :] * c2[0, 0] + st_ref[:, 1, :] * c2[0, 1]
         + st_ref[:, 2, :] * c2[0, 2] + st_ref[:, 3, :] * c2[0, 3]
         + bo1 * al_ref[0, 0])
    xn = _rms(x, nw_ref[...])

    logits = jnp.dot(xn, wr_ref[...], preferred_element_type=jnp.float32)
    scores = jax.nn.sigmoid(logits)
    biased = scores + eb_ref[...]
    iota = jax.lax.broadcasted_iota(jnp.int32, (LB, E), 1)
    m1 = jnp.max(biased, axis=1, keepdims=True)
    i1 = jnp.min(jnp.where(biased == m1, iota, E), axis=1, keepdims=True)
    oh1 = iota == i1
    tk1 = jnp.sum(jnp.where(oh1, scores, 0.0), axis=1, keepdims=True)
    b2 = jnp.where(oh1, NEG, biased)
    m2 = jnp.max(b2, axis=1, keepdims=True)
    i2 = jnp.min(jnp.where(b2 == m2, iota, E), axis=1, keepdims=True)
    oh2 = iota == i2
    tk2 = jnp.sum(jnp.where(oh2, scores, 0.0), axis=1, keepdims=True)
    den = tk1 + tk2 + 1e-8
    ew = jnp.where(oh1, tk1 / den, 0.0) + jnp.where(oh2, tk2 / den, 0.0)
    wrep = jnp.dot(ew, mexp_ref[...], preferred_element_type=jnp.float32)

    def lora3(z, sa, sb, ea, eb):
        shared = jnp.dot(jnp.dot(z, sa, preferred_element_type=jnp.float32), sb,
                         preferred_element_type=jnp.float32)
        za = jnp.dot(z, ea, preferred_element_type=jnp.float32) * wrep
        expert = jnp.dot(za, eb, preferred_element_type=jnp.float32)
        return (shared + expert) * SCALE

    gate = jnp.dot(xn, wg_ref[...], preferred_element_type=jnp.float32) \
        + lora3(xn, sag_ref[...], sbg_ref[...], eag_ref[...], ebg_ref[...])
    up = jnp.dot(xn, wu_ref[...], preferred_element_type=jnp.float32) \
        + lora3(xn, sau_ref[...], sbu_ref[...], eau_ref[...], ebu_ref[...])
    h = (gate * jax.nn.sigmoid(gate)) * up
    down = jnp.dot(h, wd_ref[...], preferred_element_type=jnp.float32) \
        + lora3(h, sad_ref[...], sbd_ref[...], ead_ref[...], ebd_ref[...])

    g = g_ref[...]
    g1 = g1_ref[...]
    po2 = po2_ref[...]
    for n in range(NS):
        o_ref[:, n, :] = (st_ref[:, 0, :] * g[n, 0] + st_ref[:, 1, :] * g[n, 1]
                          + st_ref[:, 2, :] * g[n, 2] + st_ref[:, 3, :] * g[n, 3]
                          + bo1 * g1[0, n] + down * po2[0, n])


def _qkv(st, c1, w, wq, wqp, wk, wkp, wv, cosT, sinT):
    wspec = pl.BlockSpec((D, D), lambda i: (0, 0))
    tspec = pl.BlockSpec((LB, D), lambda i: (i, 0))
    return pl.pallas_call(
        _qkv_kernel,
        grid=(NLB,),
        in_specs=[
            pl.BlockSpec((LB, NS, D), lambda i: (i, 0, 0)),
            pl.BlockSpec((1, NS), lambda i: (0, 0)),
            pl.BlockSpec((1, D), lambda i: (0, 0)),
            wspec, wspec, wspec, wspec, wspec,
            tspec, tspec,
        ],
        out_specs=[tspec, tspec, tspec],
        out_shape=[jax.ShapeDtypeStruct((L, D), jnp.float32)] * 3,
    )(st, c1, w, wq, wqp, wk, wkp, wv, cosT, sinT)


def _attention_oproj(q, k, v, wo):
    return pl.pallas_call(
        _attn_kernel,
        grid=(NLB,),
        in_specs=[
            pl.BlockSpec((LB, D), lambda i: (i, 0)),
            pl.BlockSpec((L, D), lambda i: (0, 0)),
            pl.BlockSpec((L, D), lambda i: (0, 0)),
            pl.BlockSpec((D, D), lambda i: (0, 0)),
        ],
        out_specs=pl.BlockSpec((LB, D), lambda i: (i, 0)),
        out_shape=jax.ShapeDtypeStruct((L, D), jnp.float32),
    )(q, k, v, wo)


def _ffn(st, bo1, c2, al, g, g1, po2, nw, wr, eb, mexp,
         wg, wu, wd, sag, sbg, sau, sbu, sad, sbd,
         eagf, ebgf, eauf, ebuf, eadf, ebdf):
    full = lambda a: pl.BlockSpec(a.shape, functools.partial(lambda nd, i: (0,) * nd, a.ndim))
    return pl.pallas_call(
        _ffn_kernel,
        grid=(NLB,),
        in_specs=[
            pl.BlockSpec((LB, NS, D), lambda i: (i, 0, 0)),
            pl.BlockSpec((LB, D), lambda i: (i, 0)),
            full(c2), full(al), full(g), full(g1), full(po2),
            full(nw), full(wr), full(eb), full(mexp),
            full(wg), full(wu), full(wd),
            full(sag), full(sbg), full(sau), full(sbu), full(sad), full(sbd),
            full(eagf), full(ebgf), full(eauf), full(ebuf), full(eadf), full(ebdf),
        ],
        out_specs=pl.BlockSpec((LB, NS, D), lambda i: (i, 0, 0)),
        out_shape=jax.ShapeDtypeStruct((L, NS, D), jnp.float32),
    )(st, bo1, c2, al, g, g1, po2, nw, wr, eb, mexp,
      wg, wu, wd, sag, sbg, sau, sbu, sad, sbd,
      eagf, ebgf, eauf, ebuf, eadf, ebdf)


def _rope_tables():
    pos = jnp.arange(MAXLEN, dtype=jnp.float32)
    freqs = 1.0 / (10000.0 ** (jnp.arange(0, DH, 2, dtype=jnp.float32) / DH))
    ang = pos[:, None] * freqs[None, :]
    return jnp.cos(ang), jnp.sin(ang)


def _apply_rope(x, cos, sin):
    dh = DH // 2
    c = cos[:L, :dh][None]
    s = sin[:L, :dh][None]
    x1, x2 = x[..., :dh], x[..., dh:]
    return jnp.concatenate([x1 * c - x2 * s, x2 * c + x1 * s], axis=-1)


def _hres(f1, f2):
    I2 = jnp.eye(2, dtype=jnp.float32)
    S2 = jnp.array([[0.0, 1.0], [1.0, 0.0]], dtype=jnp.float32)
    a1 = jax.nn.softmax(f1)[0]
    a2 = jax.nn.softmax(f2)[0]
    U1 = a1 * I2 + (1 - a1) * S2
    U2 = a2 * I2 + (1 - a2) * S2
    return jnp.kron(U1, U2)


def kernel(streams, W_qkv, W_o, norm1_w, norm2_w, hc1_f1, hc1_f2, hc1_pre,
           hc1_post, hc2_f1, hc2_f2, hc2_pre, hc2_post, Wg, Wu, Wd, sAg, sBg,
           sAu, sBu, sAd, sBd, eAg, eBg, eAu, eBu, eAd, eBd, Wr, expert_bias):
    cos, sin = _rope_tables()
    st = streams[0]

    Hres1 = _hres(hc1_f1, hc1_f2)
    Hres2 = _hres(hc2_f1, hc2_f2)
    pw1 = jax.nn.softmax(hc1_pre)
    pw2 = jax.nn.softmax(hc2_pre)
    po1 = jax.nn.softmax(hc1_post)
    po2 = jax.nn.softmax(hc2_post)
    c1 = pw1[None, :]
    c2 = (Hres1.T @ pw2)[None, :]
    al = (pw2 @ po1)[None, None]
    G = Hres2 @ Hres1
    g1 = (Hres2 @ po1)[None, :]

    Wq = W_qkv[:, 0:D]
    Wk = W_qkv[:, D:2 * D]
    Wv = W_qkv[:, 2 * D:3 * D]
    def _fold_p(wm):
        w4 = wm.reshape(D, H, 2, DH // 2)
        return jnp.concatenate([-w4[:, :, 1], w4[:, :, 0]], axis=2).reshape(D, D)
    WqP = _fold_p(Wq)
    WkP = _fold_p(Wk)
    cosT = jnp.tile(cos[:L, :DH // 2], (1, 2 * H))
    sinT = jnp.tile(sin[:L, :DH // 2], (1, 2 * H))

    q, k, v = _qkv(st, c1, norm1_w[None, :], Wq, WqP, Wk, WkP, Wv, cosT, sinT)
    bo1 = _attention_oproj(q, k, v, W_o)

    mexp = jnp.repeat(jnp.eye(E, dtype=jnp.float32), R, axis=1)
    eAgf = jnp.transpose(eAg, (1, 0, 2)).reshape(D, E * R)
    eBgf = eBg.reshape(E * R, DF)
    eAuf = jnp.transpose(eAu, (1, 0, 2)).reshape(D, E * R)
    eBuf = eBu.reshape(E * R, DF)
    eAdf = jnp.transpose(eAd, (1, 0, 2)).reshape(DF, E * R)
    eBdf = eBd.reshape(E * R, D)

    s2 = _ffn(st, bo1, c2, al, G, g1, po2[None, :],
              norm2_w[None, :], Wr, expert_bias[None, :], mexp,
              Wg, Wu, Wd, sAg, sBg, sAu, sBu, sAd, sBd,
              eAgf, eBgf, eAuf, eBuf, eAdf, eBdf)
    return s2[None]
